# baseline (speedup 1.0000x reference)
import jax
import jax.numpy as jnp
from jax import lax
from jax.experimental import pallas as pl
from jax.experimental.pallas import tpu as pltpu
from jax.experimental.pallas import tpu_sc as plsc

B = 16384
D = 32
L = 16
PACK = 128 // D
N = 1000000

_info = plsc.get_sparse_core_info()
NC, NS = _info.num_cores, _info.num_subcores
NW = NC * NS
BPW = B // NW
CHUNK = 128
NCHUNK = BPW // CHUNK
BLKS = CHUNK // L

NG = N // 128
NTAIL = N % 128
NTASK = 2 * NG
TPW = 4 * -(-NTASK // (4 * NW))
NCOMP = N // PACK
TAILROW = (N - NTAIL) // PACK
TAILCROWS = NTAIL * D // 128


def _relayout_body(utabT_hbm, itabT_hbm, utail_hbm, itail_hbm,
                   ucomp_hbm, icomp_hbm,
                   tbuf0, tbuf1, tbuf2, tbuf3,
                   cbuf0, cbuf1, cbuf2, cbuf3,
                   tsem0, tsem1, tsem2, tsem3,
                   osem0, osem1, osem2, osem3):
    wid = lax.axis_index("s") * NC + lax.axis_index("c")
    tbufs = (tbuf0, tbuf1, tbuf2, tbuf3)
    cbufs = (cbuf0, cbuf1, cbuf2, cbuf3)
    tsems = (tsem0, tsem1, tsem2, tsem3)
    osems = (osem0, osem1, osem2, osem3)

    @pl.when(wid == 0)
    def _():
        pltpu.sync_copy(utail_hbm, ucomp_hbm.at[pl.ds(TAILROW, TAILCROWS)])

    @pl.when(wid == 1)
    def _():
        pltpu.sync_copy(itail_hbm, icomp_hbm.at[pl.ds(TAILROW, TAILCROWS)])

    def task_of(kt):
        t = lax.rem(wid + NW * kt, NTASK)
        is_u = t < NG
        g = jnp.where(is_u, t, t - NG)
        return is_u, g

    def fire(kt, b):
        is_u, g = task_of(kt)
        off = pl.multiple_of(g * 128, 128)

        @pl.when(is_u)
        def _():
            pltpu.async_copy(utabT_hbm.at[:, pl.ds(off, 128)],
                             tbufs[b], tsems[b])

        @pl.when(jnp.logical_not(is_u))
        def _():
            pltpu.async_copy(itabT_hbm.at[:, pl.ds(off, 128)],
                             tbufs[b], tsems[b])

    def wait_t(b):
        pltpu.make_async_copy(
            utabT_hbm.at[:, pl.ds(0, 128)], tbufs[b], tsems[b]).wait()

    def wait_o(b):
        pltpu.make_async_copy(
            utabT_hbm.at[:, pl.ds(0, 128)], cbufs[b], osems[b]).wait()

    lane_iota = lax.iota(jnp.int32, L)
    rot = [(lane_iota + k) & (L - 1) for k in range(L)]
    dsta = [rot[k] * D + lane_iota for k in range(L)]

    def transpose(b, cb):
        @plsc.parallel_loop(0, (D // L) * (128 // L), 1, unroll=2)
        def _(blk):
            bd = blk >> 3
            bc = blk & 7
            rows = lane_iota + L * bd
            base = L * D * bc + L * bd
            vs = [plsc.load_gather(tbufs[b], [rows, rot[k] + L * bc])
                  for k in range(L)]
            for k in range(L):
                a = dsta[k] + base
                plsc.store_scatter(cbufs[cb], [a >> 7, a & 127], vs[k])

    def out(kt, cb):
        is_u, g = task_of(kt)
        r0 = pl.multiple_of(g * D, 8)

        @pl.when(is_u)
        def _():
            pltpu.async_copy(cbufs[cb], ucomp_hbm.at[pl.ds(r0, D)],
                             osems[cb])

        @pl.when(jnp.logical_not(is_u))
        def _():
            pltpu.async_copy(cbufs[cb], icomp_hbm.at[pl.ds(r0, D)],
                             osems[cb])

    for j in range(4):
        fire(j, j)

    def quad_body(kq, carry):
        k0 = kq * 4
        for j in range(4):
            wait_t(j)

            @pl.when(kq > 0)
            def _(j=j):
                wait_o(j)

            transpose(j, j)
            out(k0 + j, j)
            fire(k0 + 4 + j, j)
        return carry

    lax.fori_loop(0, TPW // 4, quad_body, 0)

    for j in range(4):
        wait_t(j)
        wait_o(j)


def _sc_body(uidx_hbm, iidx_hbm, utab_hbm, itab_hbm, out_hbm,
             uidx_v, iidx_v, urow_v, irow_v,
             ubuf0, ubuf1, ibuf0, ibuf1, out_v,
             usem0, usem1, isem0, isem1):
    wid = lax.axis_index("s") * NC + lax.axis_index("c")

    pltpu.sync_copy(uidx_hbm.at[wid], uidx_v)
    pltpu.sync_copy(iidx_hbm.at[wid], iidx_v)

    for k in range(BPW // L):
        sl = pl.ds(k * L, L)
        urow_v[sl] = uidx_v[sl] >> 2
        irow_v[sl] = iidx_v[sl] >> 2

    ubufs = (ubuf0, ubuf1)
    ibufs = (ibuf0, ibuf1)
    usems = (usem0, usem1)
    isems = (isem0, isem1)

    def fire(c):
        sl = pl.ds(c * CHUNK, CHUNK)
        return (
            pltpu.async_copy(utab_hbm.at[urow_v.at[sl]], ubufs[c % 2],
                             usems[c % 2]),
            pltpu.async_copy(itab_hbm.at[irow_v.at[sl]], ibufs[c % 2],
                             isems[c % 2]),
        )

    lane_iota = lax.iota(jnp.int32, L)

    def compute(c):
        ub, ib = ubufs[c % 2], ibufs[c % 2]

        def blk_body(kb, carry):
            rows = kb * L + lane_iota
            sl = pl.ds(c * CHUNK + kb * L, L)
            uoff = (uidx_v[sl] & (PACK - 1)) << 5
            ioff = (iidx_v[sl] & (PACK - 1)) << 5
            acc = jnp.zeros((L,), jnp.float32)
            for j in range(D):
                u = plsc.load_gather(ub, [rows, uoff + j])
                v = plsc.load_gather(ib, [rows, ioff + j])
                acc = acc + u * v
            out_v[sl] = 1.0 / (1.0 + jnp.exp(-acc))
            return carry

        lax.fori_loop(0, BLKS, blk_body, 0)

    cps = fire(0)
    for c in range(NCHUNK):
        nxt = fire(c + 1) if c + 1 < NCHUNK else None
        for cp in cps:
            cp.wait()
        compute(c)
        cps = nxt

    pltpu.sync_copy(out_v, out_hbm.at[pl.ds(wid * BPW, BPW)])


def _relayout(utabT, itabT, utail, itail):
    mesh = plsc.VectorSubcoreMesh(core_axis_name="c", subcore_axis_name="s")
    return pl.kernel(
        _relayout_body,
        out_type=(jax.ShapeDtypeStruct((NCOMP, 128), jnp.float32),
                  jax.ShapeDtypeStruct((NCOMP, 128), jnp.float32)),
        mesh=mesh,
        scratch_types=(
            [pltpu.VMEM((D, 128), jnp.float32)] * 8
            + [pltpu.SemaphoreType.DMA] * 8
        ),
        compiler_params=pltpu.CompilerParams(needs_layout_passes=False),
    )(utabT, itabT, utail, itail)


@jax.jit
def _run(uidx, iidx, utab4, itab4):
    mesh = plsc.VectorSubcoreMesh(core_axis_name="c", subcore_axis_name="s")
    return pl.kernel(
        _sc_body,
        out_type=jax.ShapeDtypeStruct((B,), jnp.float32),
        mesh=mesh,
        scratch_types=[
            pltpu.VMEM((BPW,), jnp.int32),
            pltpu.VMEM((BPW,), jnp.int32),
            pltpu.VMEM((BPW,), jnp.int32),
            pltpu.VMEM((BPW,), jnp.int32),
            pltpu.VMEM((CHUNK, 128), jnp.float32),
            pltpu.VMEM((CHUNK, 128), jnp.float32),
            pltpu.VMEM((CHUNK, 128), jnp.float32),
            pltpu.VMEM((CHUNK, 128), jnp.float32),
            pltpu.VMEM((BPW,), jnp.float32),
            pltpu.SemaphoreType.DMA,
            pltpu.SemaphoreType.DMA,
            pltpu.SemaphoreType.DMA,
            pltpu.SemaphoreType.DMA,
        ],
        compiler_params=pltpu.CompilerParams(needs_layout_passes=False),
    )(uidx, iidx, utab4, itab4)


@jax.jit
def kernel(user_indices, item_indices, user_table, tgt_item_table):
    uidx = user_indices.astype(jnp.int32).reshape(NW, BPW)
    iidx = item_indices.astype(jnp.int32).reshape(NW, BPW)
    utail = user_table[N - NTAIL:].reshape(TAILCROWS, 128)
    itail = tgt_item_table[N - NTAIL:].reshape(TAILCROWS, 128)
    ucomp, icomp = _relayout(user_table.T, tgt_item_table.T, utail, itail)
    return _run(uidx, iidx, ucomp, icomp)


# speedup vs baseline: 1.1174x; 1.1174x over previous
import jax
import jax.numpy as jnp
from jax import lax
from jax.experimental import pallas as pl
from jax.experimental.pallas import tpu as pltpu
from jax.experimental.pallas import tpu_sc as plsc

B = 16384
D = 32
L = 16
PACK = 128 // D
N = 1000000

_info = plsc.get_sparse_core_info()
NC, NS = _info.num_cores, _info.num_subcores
NW = NC * NS
BPW = B // NW
CHUNK = 128
NCHUNK = BPW // CHUNK
BLKS = CHUNK // L

NG = N // 128
NTAIL = N % 128
NTASK = 2 * NG
TPW = 4 * -(-NTASK // (4 * NW))
NCOMP = N // PACK
TAILROW = (N - NTAIL) // PACK
TAILCROWS = NTAIL * D // 128


def _relayout_body(utabT_hbm, itabT_hbm, utail_hbm, itail_hbm,
                   ucomp_hbm, icomp_hbm,
                   tbuf0, tbuf1, tbuf2, tbuf3,
                   cbuf0, cbuf1, cbuf2, cbuf3,
                   tsem0, tsem1, tsem2, tsem3,
                   osem0, osem1, osem2, osem3):
    wid = lax.axis_index("s") * NC + lax.axis_index("c")
    tbufs = (tbuf0, tbuf1, tbuf2, tbuf3)
    cbufs = (cbuf0, cbuf1, cbuf2, cbuf3)
    tsems = (tsem0, tsem1, tsem2, tsem3)
    osems = (osem0, osem1, osem2, osem3)

    @pl.when(wid == 0)
    def _():
        pltpu.sync_copy(utail_hbm, ucomp_hbm.at[pl.ds(TAILROW, TAILCROWS)])

    @pl.when(wid == 1)
    def _():
        pltpu.sync_copy(itail_hbm, icomp_hbm.at[pl.ds(TAILROW, TAILCROWS)])

    def task_of(kt):
        t = lax.rem(wid + NW * kt, NTASK)
        is_u = t < NG
        g = jnp.where(is_u, t, t - NG)
        return is_u, g

    def fire(kt, b):
        is_u, g = task_of(kt)
        off = pl.multiple_of(g * 128, 128)

        @pl.when(is_u)
        def _():
            pltpu.async_copy(utabT_hbm.at[:, pl.ds(off, 128)],
                             tbufs[b], tsems[b])

        @pl.when(jnp.logical_not(is_u))
        def _():
            pltpu.async_copy(itabT_hbm.at[:, pl.ds(off, 128)],
                             tbufs[b], tsems[b])

    def wait_t(b):
        pltpu.make_async_copy(
            utabT_hbm.at[:, pl.ds(0, 128)], tbufs[b], tsems[b]).wait()

    def wait_o(b):
        pltpu.make_async_copy(
            utabT_hbm.at[:, pl.ds(0, 128)], cbufs[b], osems[b]).wait()

    lane_iota = lax.iota(jnp.int32, L)
    rot = [(lane_iota + k) & (L - 1) for k in range(L)]
    dsta = [rot[k] * D + lane_iota for k in range(L)]

    def transpose(b, cb):
        @plsc.parallel_loop(0, (D // L) * (128 // L), 1, unroll=4)
        def _(blk):
            bd = blk >> 3
            bc = blk & 7
            rows = lane_iota + L * bd
            base = L * D * bc + L * bd
            vs = [plsc.load_gather(tbufs[b], [rows, rot[k] + L * bc])
                  for k in range(L)]
            for k in range(L):
                a = dsta[k] + base
                plsc.store_scatter(cbufs[cb], [a >> 7, a & 127], vs[k])

    def out(kt, cb):
        is_u, g = task_of(kt)
        r0 = pl.multiple_of(g * D, 8)

        @pl.when(is_u)
        def _():
            pltpu.async_copy(cbufs[cb], ucomp_hbm.at[pl.ds(r0, D)],
                             osems[cb])

        @pl.when(jnp.logical_not(is_u))
        def _():
            pltpu.async_copy(cbufs[cb], icomp_hbm.at[pl.ds(r0, D)],
                             osems[cb])

    fire(0, 0)

    def pair_body(kp, carry):
        k0 = kp * 2
        fire(k0 + 1, 1)
        wait_t(0)

        @pl.when(kp > 0)
        def _():
            wait_o(0)

        transpose(0, 0)
        out(k0, 0)
        fire(k0 + 2, 0)
        wait_t(1)

        @pl.when(kp > 0)
        def _():
            wait_o(1)

        transpose(1, 1)
        out(k0 + 1, 1)
        return carry

    lax.fori_loop(0, TPW // 2, pair_body, 0)

    wait_t(0)
    wait_o(0)
    wait_o(1)


def _sc_body(uidx_hbm, iidx_hbm, utab_hbm, itab_hbm, out_hbm,
             uidx_v, iidx_v, urow_v, irow_v,
             ubuf0, ubuf1, ibuf0, ibuf1, out_v,
             usem0, usem1, isem0, isem1):
    wid = lax.axis_index("s") * NC + lax.axis_index("c")

    pltpu.sync_copy(uidx_hbm.at[wid], uidx_v)
    pltpu.sync_copy(iidx_hbm.at[wid], iidx_v)

    for k in range(BPW // L):
        sl = pl.ds(k * L, L)
        urow_v[sl] = uidx_v[sl] >> 2
        irow_v[sl] = iidx_v[sl] >> 2

    ubufs = (ubuf0, ubuf1)
    ibufs = (ibuf0, ibuf1)
    usems = (usem0, usem1)
    isems = (isem0, isem1)

    def fire(c):
        sl = pl.ds(c * CHUNK, CHUNK)
        return (
            pltpu.async_copy(utab_hbm.at[urow_v.at[sl]], ubufs[c % 2],
                             usems[c % 2]),
            pltpu.async_copy(itab_hbm.at[irow_v.at[sl]], ibufs[c % 2],
                             isems[c % 2]),
        )

    lane_iota = lax.iota(jnp.int32, L)

    def compute(c):
        ub, ib = ubufs[c % 2], ibufs[c % 2]

        def blk_body(kb, carry):
            rows = kb * L + lane_iota
            sl = pl.ds(c * CHUNK + kb * L, L)
            uoff = (uidx_v[sl] & (PACK - 1)) << 5
            ioff = (iidx_v[sl] & (PACK - 1)) << 5
            acc = jnp.zeros((L,), jnp.float32)
            for j in range(D):
                u = plsc.load_gather(ub, [rows, uoff + j])
                v = plsc.load_gather(ib, [rows, ioff + j])
                acc = acc + u * v
            out_v[sl] = 1.0 / (1.0 + jnp.exp(-acc))
            return carry

        lax.fori_loop(0, BLKS, blk_body, 0)

    cps = fire(0)
    for c in range(NCHUNK):
        nxt = fire(c + 1) if c + 1 < NCHUNK else None
        for cp in cps:
            cp.wait()
        compute(c)
        cps = nxt

    pltpu.sync_copy(out_v, out_hbm.at[pl.ds(wid * BPW, BPW)])


def _relayout(utabT, itabT, utail, itail):
    mesh = plsc.VectorSubcoreMesh(core_axis_name="c", subcore_axis_name="s")
    return pl.kernel(
        _relayout_body,
        out_type=(jax.ShapeDtypeStruct((NCOMP, 128), jnp.float32),
                  jax.ShapeDtypeStruct((NCOMP, 128), jnp.float32)),
        mesh=mesh,
        scratch_types=(
            [pltpu.VMEM((D, 128), jnp.float32)] * 8
            + [pltpu.SemaphoreType.DMA] * 8
        ),
        compiler_params=pltpu.CompilerParams(needs_layout_passes=False),
    )(utabT, itabT, utail, itail)


@jax.jit
def _run(uidx, iidx, utab4, itab4):
    mesh = plsc.VectorSubcoreMesh(core_axis_name="c", subcore_axis_name="s")
    return pl.kernel(
        _sc_body,
        out_type=jax.ShapeDtypeStruct((B,), jnp.float32),
        mesh=mesh,
        scratch_types=[
            pltpu.VMEM((BPW,), jnp.int32),
            pltpu.VMEM((BPW,), jnp.int32),
            pltpu.VMEM((BPW,), jnp.int32),
            pltpu.VMEM((BPW,), jnp.int32),
            pltpu.VMEM((CHUNK, 128), jnp.float32),
            pltpu.VMEM((CHUNK, 128), jnp.float32),
            pltpu.VMEM((CHUNK, 128), jnp.float32),
            pltpu.VMEM((CHUNK, 128), jnp.float32),
            pltpu.VMEM((BPW,), jnp.float32),
            pltpu.SemaphoreType.DMA,
            pltpu.SemaphoreType.DMA,
            pltpu.SemaphoreType.DMA,
            pltpu.SemaphoreType.DMA,
        ],
        compiler_params=pltpu.CompilerParams(needs_layout_passes=False),
    )(uidx, iidx, utab4, itab4)


@jax.jit
def kernel(user_indices, item_indices, user_table, tgt_item_table):
    uidx = user_indices.astype(jnp.int32).reshape(NW, BPW)
    iidx = item_indices.astype(jnp.int32).reshape(NW, BPW)
    utail = user_table[N - NTAIL:].reshape(TAILCROWS, 128)
    itail = tgt_item_table[N - NTAIL:].reshape(TAILCROWS, 128)
    ucomp, icomp = _relayout(user_table.T, tgt_item_table.T, utail, itail)
    return _run(uidx, iidx, ucomp, icomp)


# speedup vs baseline: 1.1221x; 1.0042x over previous
import jax
import jax.numpy as jnp
from jax import lax
from jax.experimental import pallas as pl
from jax.experimental.pallas import tpu as pltpu
from jax.experimental.pallas import tpu_sc as plsc

B = 16384
D = 32
L = 16
PACK = 128 // D
N = 1000000

_info = plsc.get_sparse_core_info()
NC, NS = _info.num_cores, _info.num_subcores
NW = NC * NS
BPW = B // NW
CHUNK = 128
NCHUNK = BPW // CHUNK
BLKS = CHUNK // L

NG = N // 128
NTAIL = N % 128
NTASK = 2 * NG
TPW = 2 * -(-NTASK // (2 * NW))
NCOMP = N // PACK
TAILROW = (N - NTAIL) // PACK
TAILCROWS = NTAIL * D // 128


def _relayout_body(utabT_hbm, itabT_hbm, utail_hbm, itail_hbm,
                   ucomp_hbm, icomp_hbm,
                   tbuf0, tbuf1, cbuf0, cbuf1,
                   tsem0, tsem1, osem0, osem1):
    wid = lax.axis_index("s") * NC + lax.axis_index("c")
    tbufs = (tbuf0, tbuf1)
    cbufs = (cbuf0, cbuf1)
    tsems = (tsem0, tsem1)
    osems = (osem0, osem1)

    @pl.when(wid == 0)
    def _():
        pltpu.sync_copy(utail_hbm, ucomp_hbm.at[pl.ds(TAILROW, TAILCROWS)])

    @pl.when(wid == 1)
    def _():
        pltpu.sync_copy(itail_hbm, icomp_hbm.at[pl.ds(TAILROW, TAILCROWS)])

    def task_of(kt):
        t = lax.rem(wid + NW * kt, NTASK)
        is_u = t < NG
        g = jnp.where(is_u, t, t - NG)
        return is_u, g

    def fire(kt, b):
        is_u, g = task_of(kt)
        off = pl.multiple_of(g * 128, 128)

        @pl.when(is_u)
        def _():
            pltpu.async_copy(utabT_hbm.at[:, pl.ds(off, 128)],
                             tbufs[b], tsems[b])

        @pl.when(jnp.logical_not(is_u))
        def _():
            pltpu.async_copy(itabT_hbm.at[:, pl.ds(off, 128)],
                             tbufs[b], tsems[b])

    def wait_t(b):
        pltpu.make_async_copy(
            utabT_hbm.at[:, pl.ds(0, 128)], tbufs[b], tsems[b]).wait()

    def wait_o(b):
        pltpu.make_async_copy(
            utabT_hbm.at[:, pl.ds(0, 128)], cbufs[b], osems[b]).wait()

    lane_iota = lax.iota(jnp.int32, L)
    rot = [(lane_iota + k) & (L - 1) for k in range(L)]
    dsta = [rot[k] * D + lane_iota for k in range(L)]

    def transpose(b, cb):
        @plsc.parallel_loop(0, (D // L) * (128 // L), 1, unroll=4)
        def _(blk):
            bd = blk >> 3
            bc = blk & 7
            rows = lane_iota + L * bd
            base = L * D * bc + L * bd
            vs = [plsc.load_gather(tbufs[b], [rows, rot[k] + L * bc])
                  for k in range(L)]
            for k in range(L):
                a = dsta[k] + base
                plsc.store_scatter(cbufs[cb], [a >> 7, a & 127], vs[k])

    def out(kt, cb):
        is_u, g = task_of(kt)
        r0 = pl.multiple_of(g * D, 8)

        @pl.when(is_u)
        def _():
            pltpu.async_copy(cbufs[cb], ucomp_hbm.at[pl.ds(r0, D)],
                             osems[cb])

        @pl.when(jnp.logical_not(is_u))
        def _():
            pltpu.async_copy(cbufs[cb], icomp_hbm.at[pl.ds(r0, D)],
                             osems[cb])

    fire(0, 0)

    def pair_body(kp, carry):
        k0 = kp * 2
        fire(k0 + 1, 1)
        wait_t(0)

        @pl.when(kp > 0)
        def _():
            wait_o(0)

        transpose(0, 0)
        out(k0, 0)
        fire(k0 + 2, 0)
        wait_t(1)

        @pl.when(kp > 0)
        def _():
            wait_o(1)

        transpose(1, 1)
        out(k0 + 1, 1)
        return carry

    lax.fori_loop(0, TPW // 2, pair_body, 0)

    wait_t(0)
    wait_o(0)
    wait_o(1)


def _sc_body(uidx_hbm, iidx_hbm, utab_hbm, itab_hbm, out_hbm,
             uidx_v, iidx_v, urow_v, irow_v,
             ubuf0, ubuf1, ibuf0, ibuf1, out_v,
             usem0, usem1, isem0, isem1):
    wid = lax.axis_index("s") * NC + lax.axis_index("c")

    pltpu.sync_copy(uidx_hbm.at[wid], uidx_v)
    pltpu.sync_copy(iidx_hbm.at[wid], iidx_v)

    for k in range(BPW // L):
        sl = pl.ds(k * L, L)
        urow_v[sl] = uidx_v[sl] >> 2
        irow_v[sl] = iidx_v[sl] >> 2

    ubufs = (ubuf0, ubuf1)
    ibufs = (ibuf0, ibuf1)
    usems = (usem0, usem1)
    isems = (isem0, isem1)

    def fire(c):
        sl = pl.ds(c * CHUNK, CHUNK)
        return (
            pltpu.async_copy(utab_hbm.at[urow_v.at[sl]], ubufs[c % 2],
                             usems[c % 2]),
            pltpu.async_copy(itab_hbm.at[irow_v.at[sl]], ibufs[c % 2],
                             isems[c % 2]),
        )

    lane_iota = lax.iota(jnp.int32, L)

    def compute(c):
        ub, ib = ubufs[c % 2], ibufs[c % 2]

        def blk_body(kb, carry):
            rows = kb * L + lane_iota
            sl = pl.ds(c * CHUNK + kb * L, L)
            uoff = (uidx_v[sl] & (PACK - 1)) << 5
            ioff = (iidx_v[sl] & (PACK - 1)) << 5
            acc = jnp.zeros((L,), jnp.float32)
            for j in range(D):
                u = plsc.load_gather(ub, [rows, uoff + j])
                v = plsc.load_gather(ib, [rows, ioff + j])
                acc = acc + u * v
            out_v[sl] = 1.0 / (1.0 + jnp.exp(-acc))
            return carry

        lax.fori_loop(0, BLKS, blk_body, 0)

    cps = fire(0)
    for c in range(NCHUNK):
        nxt = fire(c + 1) if c + 1 < NCHUNK else None
        for cp in cps:
            cp.wait()
        compute(c)
        cps = nxt

    pltpu.sync_copy(out_v, out_hbm.at[pl.ds(wid * BPW, BPW)])


def _relayout(utabT, itabT, utail, itail):
    mesh = plsc.VectorSubcoreMesh(core_axis_name="c", subcore_axis_name="s")
    return pl.kernel(
        _relayout_body,
        out_type=(jax.ShapeDtypeStruct((NCOMP, 128), jnp.float32),
                  jax.ShapeDtypeStruct((NCOMP, 128), jnp.float32)),
        mesh=mesh,
        scratch_types=(
            [pltpu.VMEM((D, 128), jnp.float32)] * 4
            + [pltpu.SemaphoreType.DMA] * 4
        ),
        compiler_params=pltpu.CompilerParams(needs_layout_passes=False),
    )(utabT, itabT, utail, itail)


@jax.jit
def _run(uidx, iidx, utab4, itab4):
    mesh = plsc.VectorSubcoreMesh(core_axis_name="c", subcore_axis_name="s")
    return pl.kernel(
        _sc_body,
        out_type=jax.ShapeDtypeStruct((B,), jnp.float32),
        mesh=mesh,
        scratch_types=[
            pltpu.VMEM((BPW,), jnp.int32),
            pltpu.VMEM((BPW,), jnp.int32),
            pltpu.VMEM((BPW,), jnp.int32),
            pltpu.VMEM((BPW,), jnp.int32),
            pltpu.VMEM((CHUNK, 128), jnp.float32),
            pltpu.VMEM((CHUNK, 128), jnp.float32),
            pltpu.VMEM((CHUNK, 128), jnp.float32),
            pltpu.VMEM((CHUNK, 128), jnp.float32),
            pltpu.VMEM((BPW,), jnp.float32),
            pltpu.SemaphoreType.DMA,
            pltpu.SemaphoreType.DMA,
            pltpu.SemaphoreType.DMA,
            pltpu.SemaphoreType.DMA,
        ],
        compiler_params=pltpu.CompilerParams(needs_layout_passes=False),
    )(uidx, iidx, utab4, itab4)


@jax.jit
def kernel(user_indices, item_indices, user_table, tgt_item_table):
    uidx = user_indices.astype(jnp.int32).reshape(NW, BPW)
    iidx = item_indices.astype(jnp.int32).reshape(NW, BPW)
    utail = user_table[N - NTAIL:].reshape(TAILCROWS, 128)
    itail = tgt_item_table[N - NTAIL:].reshape(TAILCROWS, 128)
    ucomp, icomp = _relayout(user_table.T, tgt_item_table.T, utail, itail)
    return _run(uidx, iidx, ucomp, icomp)
